# baseline (speedup 1.0000x reference)
import functools

import jax
import jax.numpy as jnp
from jax import lax
from jax.experimental import pallas as pl
from jax.experimental.pallas import tpu as pltpu
from jax.experimental.pallas import tpu_sc as plsc

B = 16384
EMB = 32
NTAB = 5
NC = 2
NS = 16
NW = NC * NS
B_PER_W = B // NW
CHUNK = 256


def _sc_lookup_concat(authdir, year, actor, rated, genre,
                      W_authdir, W_year, W_actor, W_rated, W_genre):
    mesh = plsc.VectorSubcoreMesh(core_axis_name="c", subcore_axis_name="s",
                                  num_cores=NC, num_subcores=NS)

    @functools.partial(
        pl.kernel,
        mesh=mesh,
        out_type=jax.ShapeDtypeStruct((B, NTAB * EMB), jnp.float32),
        scratch_types=(
            [pltpu.VMEM((B_PER_W,), jnp.int32) for _ in range(NTAB)]
            + [pltpu.VMEM((B_PER_W, EMB), jnp.float32) for _ in range(NTAB)]
            + [pltpu.VMEM_SHARED((NS * CHUNK, NTAB * EMB), jnp.float32)]
            + [pltpu.SemaphoreType.DMA, pltpu.SemaphoreType.DMA,
               pltpu.SemaphoreType.DMA]
        ),
        compiler_params=pltpu.CompilerParams(use_tc_tiling_on_sc=False),
    )
    def body(a_i, y_i, ac_i, r_i, g_i, Wa, Wy, Wac, Wr, Wg, out,
             i0, i1, i2, i3, i4, r0, r1, r2, r3, r4, out_s, sem_i, sem_g,
             sem_s):
        sid = lax.axis_index("s")
        wid = sid * NC + lax.axis_index("c")
        base = wid * B_PER_W
        sbase = sid * CHUNK
        idx_hbm = [a_i, y_i, ac_i, r_i, g_i]
        tabs = [Wa, Wy, Wac, Wr, Wg]
        idx_v = [i0, i1, i2, i3, i4]
        row_v = [r0, r1, r2, r3, r4]

        idx_copies = [
            pltpu.async_copy(idx_hbm[c].at[pl.ds(base, B_PER_W)], idx_v[c], sem_i)
            for c in range(NTAB)
        ]
        for c in range(NTAB):
            idx_copies[c].wait()
        for h in range(B_PER_W // CHUNK):
            gathers = [
                pltpu.async_copy(
                    tabs[c].at[idx_v[c].at[pl.ds(h * CHUNK, CHUNK)]],
                    row_v[c].at[pl.ds(0, CHUNK), :], sem_g)
                for c in range(NTAB)
            ]
            spreads = []
            for c in range(NTAB):
                gathers[c].wait()
                spreads.append(
                    pltpu.async_copy(
                        row_v[c].at[pl.ds(0, CHUNK), :],
                        out_s.at[pl.ds(sbase, CHUNK), pl.ds(c * EMB, EMB)],
                        sem_s))
            for c in range(NTAB):
                spreads[c].wait()
            pltpu.sync_copy(out_s.at[pl.ds(sbase, CHUNK), :],
                            out.at[pl.ds(base + h * CHUNK, CHUNK), :])

    return body(authdir, year, actor, rated, genre,
                W_authdir, W_year, W_actor, W_rated, W_genre)


def kernel(authdir, year, actor, rated, genre,
           W_authdir, W_year, W_actor, W_rated, W_genre):
    return _sc_lookup_concat(authdir, year, actor, rated, genre,
                             W_authdir, W_year, W_actor, W_rated, W_genre)


# speedup vs baseline: 1.2920x; 1.2920x over previous
import functools

import jax
import jax.numpy as jnp
from jax import lax
from jax.experimental import pallas as pl
from jax.experimental.pallas import tpu as pltpu
from jax.experimental.pallas import tpu_sc as plsc

B = 16384
EMB = 32
NTAB = 5
NC = 2
NS = 16
NW = NC * NS
B_PER_W = B // NW
RPP = 256
NPASS = B_PER_W // RPP
CH = 16


def kernel(authdir, year, actor, rated, genre,
           W_authdir, W_year, W_actor, W_rated, W_genre):
    mesh = plsc.VectorSubcoreMesh(core_axis_name="c", subcore_axis_name="s",
                                  num_cores=NC, num_subcores=NS)

    @functools.partial(
        pl.kernel,
        mesh=mesh,
        out_type=jax.ShapeDtypeStruct((B, NTAB * EMB), jnp.float32),
        compiler_params=pltpu.CompilerParams(needs_layout_passes=False),
        scratch_types=(
            pltpu.VMEM((RPP,), jnp.int32),
            pltpu.VMEM((RPP, NTAB * EMB), jnp.float32),
            pltpu.SemaphoreType.DMA,
        ),
    )
    def body(a_i, y_i, ac_i, r_i, g_i, Wa, Wy, Wac, Wr, Wg, out,
             idx_v, buf, sem):
        wid = lax.axis_index("s") * NC + lax.axis_index("c")
        base = wid * B_PER_W
        idx_hbm = [a_i, y_i, ac_i, r_i, g_i]
        tabs = [Wa, Wy, Wac, Wr, Wg]
        lanes = lax.iota(jnp.int32, 16)
        imin = jnp.int32(-2147483648)

        for p in range(NPASS):
            pbase = base + p * RPP
            for c in range(NTAB):
                pltpu.sync_copy(idx_hbm[c].at[pl.ds(pbase, RPP)], idx_v)

                @plsc.parallel_loop(0, RPP, CH)
                def _(j):
                    v = idx_v[pl.ds(j, CH)]
                    copies = []
                    for t in range(CH):
                        i = jnp.max(jnp.where(lanes == t, v, imin))
                        copies.append(pltpu.async_copy(
                            tabs[c].at[i],
                            buf.at[j + t, pl.ds(c * EMB, EMB)], sem))
                    for cp in copies:
                        cp.wait()

            pltpu.sync_copy(buf, out.at[pl.ds(pbase, RPP), :])

    return body(authdir, year, actor, rated, genre,
                W_authdir, W_year, W_actor, W_rated, W_genre)
